# baseline (speedup 1.0000x reference)
import jax
import jax.numpy as jnp
from jax.experimental import pallas as pl
from jax.experimental.pallas import tpu as pltpu

_NUM_JOINTS = 25
_NUM_ACTORS = 8
_NUM_CLASSES = 6
_FEAT = 2048

_BB = 256
_SH = 40
_SH5 = 24
_H0 = 8
_NR = _H0 + _BB * _SH + 8
_NR5 = _H0 + _BB * _SH5 + 8


def _conv_kernel(xr_ref, wfm_ref, b2t_ref, w3b_ref, b3t_ref, w4b_ref, b4t_ref,
                 w5b_ref, b5t_ref, se16e_ref, se16o_ref, se8e_ref, se8o_ref,
                 out_ref, buf3, buf4, buf5):
    f32 = jnp.float32
    bf16 = jnp.bfloat16

    for buf, ss, nv in ((buf3, _SH, 32), (buf4, _SH, 32), (buf5, _SH5, 16)):
        buf[0:_H0, :] = jnp.zeros((_H0, buf.shape[1]), bf16)
        for s in range(_BB):
            r = _H0 + s * ss + nv
            buf[r: r + 8, :] = jnp.zeros((8, buf.shape[1]), bf16)

    for g in range(_BB // 8):
        pa = jnp.dot(wfm_ref[...], xr_ref[0, :, g * 2048: g * 2048 + 2048],
                     preferred_element_type=f32)
        pa = (pa + b2t_ref[...]).astype(bf16)
        for s0 in range(8):
            s = g * 8 + s0
            buf3[_H0 + s * _SH: _H0 + s * _SH + 32, :] = pa[:, s0 * 256: s0 * 256 + 256]

    for c in range(_BB // 4):
        lo = _H0 + c * 4 * _SH
        m = 4 * _SH - 8
        y3 = (jnp.dot(buf3[lo - 1: lo - 1 + m, :], w3b_ref[0],
                      preferred_element_type=f32)
              + jnp.dot(buf3[lo: lo + m, :], w3b_ref[1],
                        preferred_element_type=f32)
              + jnp.dot(buf3[lo + 1: lo + 1 + m, :], w3b_ref[2],
                        preferred_element_type=f32)
              + b3t_ref[...]).astype(bf16)
        for s in range(4):
            buf4[lo + s * _SH: lo + s * _SH + 32, :] = y3[s * _SH: s * _SH + 32, :]

    mcat = []
    for c in range(_BB // 4):
        lo = _H0 + c * 4 * _SH
        m = 4 * _SH - 8
        y4 = (jnp.dot(buf4[lo - 1: lo - 1 + m, :], w4b_ref[0],
                      preferred_element_type=f32)
              + jnp.dot(buf4[lo: lo + m, :], w4b_ref[1],
                        preferred_element_type=f32)
              + jnp.dot(buf4[lo + 1: lo + 1 + m, :], w4b_ref[2],
                        preferred_element_type=f32)
              + b4t_ref[...])
        mw = jnp.concatenate(
            [jnp.maximum(y4[:, 64 * a: 64 * a + 32], y4[:, 64 * a + 32: 64 * a + 64])
             for a in range(4)], axis=1)
        for s in range(4):
            mcat.append(mw[s * _SH: s * _SH + 32, :])
    mcat = jnp.concatenate(mcat, axis=1)
    p4a = jnp.maximum(
        jnp.dot(se16e_ref[...], mcat, preferred_element_type=f32),
        jnp.dot(se16o_ref[...], mcat, preferred_element_type=f32))
    p4a = p4a.astype(bf16)
    for s in range(_BB):
        buf5[_H0 + s * _SH5: _H0 + s * _SH5 + 16, :] = p4a[:, s * 128: s * 128 + 128]

    m5cat = []
    for g in range(_BB // 8):
        lo = _H0 + g * 8 * _SH5
        m = 8 * _SH5 - 8
        y5 = (jnp.dot(buf5[lo - 1: lo - 1 + m, :], w5b_ref[0],
                      preferred_element_type=f32)
              + jnp.dot(buf5[lo: lo + m, :], w5b_ref[1],
                        preferred_element_type=f32)
              + jnp.dot(buf5[lo + 1: lo + 1 + m, :], w5b_ref[2],
                        preferred_element_type=f32)
              + b5t_ref[...])
        m5 = jnp.concatenate([jnp.maximum(y5[:, 0:128], y5[:, 128:256]),
                              jnp.maximum(y5[:, 256:384], y5[:, 384:512])],
                             axis=1)
        m5cat.extend(m5[s * _SH5: s * _SH5 + 16, :] for s in range(8))
    m5cat = jnp.concatenate(m5cat, axis=1)
    p5a = jnp.maximum(
        jnp.dot(se8e_ref[...], m5cat, preferred_element_type=f32),
        jnp.dot(se8o_ref[...], m5cat, preferred_element_type=f32))
    p5a = jnp.maximum(p5a, 0.0)
    for s in range(_BB):
        for w2 in range(2):
            c0 = s * 256 + w2 * 128
            out_ref[s, w2 * 8: w2 * 8 + 8, :] = p5a[:, c0: c0 + 128]


def _fc_head_kernel(x_ref, w1_ref, b1_ref, w2_ref, b2_ref, o_ref):
    h = jnp.dot(x_ref[...], w1_ref[...], preferred_element_type=jnp.float32)
    h = h + b1_ref[...]
    y = jnp.dot(h, w2_ref[...], preferred_element_type=jnp.float32)
    o_ref[...] = y + b2_ref[...]


def _conv_features(xr, wfm, b2t, w3b, b3t, w4b, b4t, w5b, b5t,
                   se16e, se16o, se8e, se8o):
    nb = xr.shape[0]
    return pl.pallas_call(
        _conv_kernel,
        out_shape=jax.ShapeDtypeStruct((nb * _BB, 16, 128), jnp.float32),
        grid=(nb,),
        in_specs=[
            pl.BlockSpec((1, 8, _BB * 256), lambda i: (i, 0, 0)),
            pl.BlockSpec((32, 8), lambda i: (0, 0)),
            pl.BlockSpec((32, 2048), lambda i: (0, 0)),
            pl.BlockSpec((3, 256, 512), lambda i: (0, 0, 0)),
            pl.BlockSpec((1, 512), lambda i: (0, 0)),
            pl.BlockSpec((3, 512, 256), lambda i: (0, 0, 0)),
            pl.BlockSpec((1, 256), lambda i: (0, 0)),
            pl.BlockSpec((3, 128, 512), lambda i: (0, 0, 0)),
            pl.BlockSpec((1, 512), lambda i: (0, 0)),
            pl.BlockSpec((16, 32), lambda i: (0, 0)),
            pl.BlockSpec((16, 32), lambda i: (0, 0)),
            pl.BlockSpec((8, 16), lambda i: (0, 0)),
            pl.BlockSpec((8, 16), lambda i: (0, 0)),
        ],
        out_specs=pl.BlockSpec((_BB, 16, 128), lambda i: (i, 0, 0)),
        scratch_shapes=[
            pltpu.VMEM((_NR, 256), jnp.bfloat16),
            pltpu.VMEM((_NR, 512), jnp.bfloat16),
            pltpu.VMEM((_NR5, 128), jnp.bfloat16),
        ],
        compiler_params=pltpu.CompilerParams(dimension_semantics=("parallel",)),
    )(xr, wfm, b2t, w3b, b3t, w4b, b4t, w5b, b5t, se16e, se16o, se8e, se8o)


def _fc_head(person, w1t, b1f, w2p, b2f):
    Bp = person.shape[0]
    bm = next(d for d in (256, 128, 64, 32, 16, 8) if Bp % d == 0)
    return pl.pallas_call(
        _fc_head_kernel,
        out_shape=jax.ShapeDtypeStruct((Bp, 8), jnp.float32),
        grid=(Bp // bm,),
        in_specs=[
            pl.BlockSpec((bm, _FEAT), lambda i: (i, 0)),
            pl.BlockSpec((_FEAT, 256), lambda i: (0, 0)),
            pl.BlockSpec((1, 256), lambda i: (0, 0)),
            pl.BlockSpec((256, 8), lambda i: (0, 0)),
            pl.BlockSpec((1, 8), lambda i: (0, 0)),
        ],
        out_specs=pl.BlockSpec((bm, 8), lambda i: (i, 0)),
        compiler_params=pltpu.CompilerParams(dimension_semantics=("parallel",)),
    )(person, w1t, b1f, w2p, b2f)


def _tridiag(wt, cin, cout, nw):
    f32 = jnp.float32
    out = jnp.zeros((nw * cin, nw * cout), f32)
    ii = jnp.arange(nw)
    for t in range(3):
        e = ((ii[:, None] - ii[None, :]) == (t - 1)).astype(f32)
        out = out + jnp.kron(e, wt[t].astype(f32))
    return out.astype(jnp.bfloat16)


@jax.jit
def _forward(X, wfa, b2m, w3, b3, w4, b4, w5, b5,
             se16e, se16o, se8e, se8o, w1t, b1f, w2p, b2f):
    f32 = jnp.float32
    x = X.reshape(-1, 2, _NUM_JOINTS, _NUM_ACTORS).astype(f32)
    B = x.shape[0]
    Bp = ((B + _BB - 1) // _BB) * _BB
    nb = Bp // _BB

    xpad = jnp.pad(x, ((0, Bp - B), (0, 0), (1, 8), (0, 0)))
    taps = [xpad[:, kind, kh: kh + 32, :].reshape(Bp, 256)
            for kind in range(2) for kh in range(3)]
    xr = jnp.stack(taps, axis=1)
    xr = xr.reshape(nb, _BB, 6, 256).transpose(0, 2, 1, 3).reshape(nb, 6, _BB * 256)
    xr = jnp.pad(xr, ((0, 0), (0, 2), (0, 0))).astype(jnp.bfloat16)

    wfm = jnp.pad(jnp.transpose(wfa[..., 0], (2, 1, 0)).reshape(32, 6),
                  ((0, 0), (0, 2))).astype(jnp.bfloat16)
    b2t = jnp.tile(jnp.repeat(b2m, 8, axis=1), (1, 8))
    w3b = jnp.stack([_tridiag(w3[:, kh * 32: kh * 32 + 32, :], 32, 64, 8)
                     for kh in range(3)])
    pr = (jnp.arange(256) % 8) * 32 + jnp.arange(256) // 8
    w3b = w3b[:, pr, :]
    w4b = jnp.stack([_tridiag(w4[:, kh * 64: kh * 64 + 64, :], 64, 32, 8)
                     for kh in range(3)])
    w5b = jnp.stack([_tridiag(w5[:, kh * 32: kh * 32 + 32, :], 32, 128, 4)
                     for kh in range(3)])
    b3t = jnp.tile(b3, (1, 8))
    b4t = jnp.tile(b4, (1, 8))
    b5t = jnp.tile(b5, (1, 4))

    feats = _conv_features(xr, wfm, b2t, w3b, b3t, w4b, b4t, w5b, b5t,
                           se16e, se16o, se8e, se8o)
    person = feats.reshape(Bp, _FEAT)
    out = _fc_head(person, w1t, b1f, w2p[:, 0:8], b2f[:, 0:8])
    return out[:B, :_NUM_CLASSES]


def kernel(X, wfa, b2m, w3, b3, w4, b4, w5, b5,
           se16e, se16o, se8e, se8o, w1t, b1f, w2p, b2f):
    return _forward(X, wfa, b2m, w3, b3, w4, b4, w5, b5,
                    se16e, se16o, se8e, se8o, w1t, b1f, w2p, b2f)


# speedup vs baseline: 1.0258x; 1.0258x over previous
import jax
import jax.numpy as jnp
from jax.experimental import pallas as pl
from jax.experimental.pallas import tpu as pltpu

_NUM_JOINTS = 25
_NUM_ACTORS = 8
_NUM_CLASSES = 6
_FEAT = 2048

_BB = 128
_SH = 40
_SH5 = 24
_H0 = 8
_NR = _H0 + _BB * _SH + 8
_NR5 = _H0 + _BB * _SH5 + 8


def _conv_kernel(xr_ref, wfm_ref, b2t_ref, w3b_ref, b3t_ref, w4b_ref, b4t_ref,
                 w5b_ref, b5t_ref, se16e_ref, se16o_ref, se8e_ref, se8o_ref,
                 out_ref, buf3, buf4, buf5):
    f32 = jnp.float32
    bf16 = jnp.bfloat16

    for buf, ss, nv in ((buf3, _SH, 32), (buf4, _SH, 32), (buf5, _SH5, 16)):
        buf[0:_H0, :] = jnp.zeros((_H0, buf.shape[1]), bf16)
        for s in range(_BB):
            r = _H0 + s * ss + nv
            buf[r: r + 8, :] = jnp.zeros((8, buf.shape[1]), bf16)

    for g in range(_BB // 8):
        pa = jnp.dot(wfm_ref[...], xr_ref[0, :, g * 2048: g * 2048 + 2048],
                     preferred_element_type=f32)
        pa = (pa + b2t_ref[...]).astype(bf16)
        for s0 in range(8):
            s = g * 8 + s0
            buf3[_H0 + s * _SH: _H0 + s * _SH + 32, :] = pa[:, s0 * 256: s0 * 256 + 256]

    for c in range(_BB // 4):
        lo = _H0 + c * 4 * _SH
        m = 4 * _SH - 8
        y3 = (jnp.dot(buf3[lo - 1: lo - 1 + m, :], w3b_ref[0],
                      preferred_element_type=f32)
              + jnp.dot(buf3[lo: lo + m, :], w3b_ref[1],
                        preferred_element_type=f32)
              + jnp.dot(buf3[lo + 1: lo + 1 + m, :], w3b_ref[2],
                        preferred_element_type=f32)
              + b3t_ref[...]).astype(bf16)
        for s in range(4):
            buf4[lo + s * _SH: lo + s * _SH + 32, :] = y3[s * _SH: s * _SH + 32, :]

    mcat = []
    for c in range(_BB // 4):
        lo = _H0 + c * 4 * _SH
        m = 4 * _SH - 8
        y4 = (jnp.dot(buf4[lo - 1: lo - 1 + m, :], w4b_ref[0],
                      preferred_element_type=f32)
              + jnp.dot(buf4[lo: lo + m, :], w4b_ref[1],
                        preferred_element_type=f32)
              + jnp.dot(buf4[lo + 1: lo + 1 + m, :], w4b_ref[2],
                        preferred_element_type=f32)
              + b4t_ref[...])
        mw = jnp.concatenate(
            [jnp.maximum(y4[:, 64 * a: 64 * a + 32], y4[:, 64 * a + 32: 64 * a + 64])
             for a in range(4)], axis=1)
        for s in range(4):
            mcat.append(mw[s * _SH: s * _SH + 32, :])
    mcat = jnp.concatenate(mcat, axis=1)
    p4a = jnp.maximum(
        jnp.dot(se16e_ref[...], mcat, preferred_element_type=f32),
        jnp.dot(se16o_ref[...], mcat, preferred_element_type=f32))
    p4a = p4a.astype(bf16)
    for s in range(_BB):
        buf5[_H0 + s * _SH5: _H0 + s * _SH5 + 16, :] = p4a[:, s * 128: s * 128 + 128]

    m5cat = []
    for g in range(_BB // 8):
        lo = _H0 + g * 8 * _SH5
        m = 8 * _SH5 - 8
        y5 = (jnp.dot(buf5[lo - 1: lo - 1 + m, :], w5b_ref[0],
                      preferred_element_type=f32)
              + jnp.dot(buf5[lo: lo + m, :], w5b_ref[1],
                        preferred_element_type=f32)
              + jnp.dot(buf5[lo + 1: lo + 1 + m, :], w5b_ref[2],
                        preferred_element_type=f32)
              + b5t_ref[...])
        m5 = jnp.concatenate([jnp.maximum(y5[:, 0:128], y5[:, 128:256]),
                              jnp.maximum(y5[:, 256:384], y5[:, 384:512])],
                             axis=1)
        m5cat.extend(m5[s * _SH5: s * _SH5 + 16, :] for s in range(8))
    m5cat = jnp.concatenate(m5cat, axis=1)
    p5a = jnp.maximum(
        jnp.dot(se8e_ref[...], m5cat, preferred_element_type=f32),
        jnp.dot(se8o_ref[...], m5cat, preferred_element_type=f32))
    p5a = jnp.maximum(p5a, 0.0)
    for s in range(_BB):
        for w2 in range(2):
            c0 = s * 256 + w2 * 128
            out_ref[s, w2 * 8: w2 * 8 + 8, :] = p5a[:, c0: c0 + 128]


def _fc_head_kernel(x_ref, w1_ref, b1_ref, w2_ref, b2_ref, o_ref):
    h = jnp.dot(x_ref[...], w1_ref[...], preferred_element_type=jnp.float32)
    h = h + b1_ref[...]
    y = jnp.dot(h, w2_ref[...], preferred_element_type=jnp.float32)
    o_ref[...] = y + b2_ref[...]


def _conv_features(xr, wfm, b2t, w3b, b3t, w4b, b4t, w5b, b5t,
                   se16e, se16o, se8e, se8o):
    nb = xr.shape[0]
    return pl.pallas_call(
        _conv_kernel,
        out_shape=jax.ShapeDtypeStruct((nb * _BB, 16, 128), jnp.float32),
        grid=(nb,),
        in_specs=[
            pl.BlockSpec((1, 8, _BB * 256), lambda i: (i, 0, 0)),
            pl.BlockSpec((32, 8), lambda i: (0, 0)),
            pl.BlockSpec((32, 2048), lambda i: (0, 0)),
            pl.BlockSpec((3, 256, 512), lambda i: (0, 0, 0)),
            pl.BlockSpec((1, 512), lambda i: (0, 0)),
            pl.BlockSpec((3, 512, 256), lambda i: (0, 0, 0)),
            pl.BlockSpec((1, 256), lambda i: (0, 0)),
            pl.BlockSpec((3, 128, 512), lambda i: (0, 0, 0)),
            pl.BlockSpec((1, 512), lambda i: (0, 0)),
            pl.BlockSpec((16, 32), lambda i: (0, 0)),
            pl.BlockSpec((16, 32), lambda i: (0, 0)),
            pl.BlockSpec((8, 16), lambda i: (0, 0)),
            pl.BlockSpec((8, 16), lambda i: (0, 0)),
        ],
        out_specs=pl.BlockSpec((_BB, 16, 128), lambda i: (i, 0, 0)),
        scratch_shapes=[
            pltpu.VMEM((_NR, 256), jnp.bfloat16),
            pltpu.VMEM((_NR, 512), jnp.bfloat16),
            pltpu.VMEM((_NR5, 128), jnp.bfloat16),
        ],
        compiler_params=pltpu.CompilerParams(dimension_semantics=("parallel",)),
    )(xr, wfm, b2t, w3b, b3t, w4b, b4t, w5b, b5t, se16e, se16o, se8e, se8o)


def _fc_head(person, w1t, b1f, w2p, b2f):
    Bp = person.shape[0]
    bm = next(d for d in (256, 128, 64, 32, 16, 8) if Bp % d == 0)
    return pl.pallas_call(
        _fc_head_kernel,
        out_shape=jax.ShapeDtypeStruct((Bp, 8), jnp.float32),
        grid=(Bp // bm,),
        in_specs=[
            pl.BlockSpec((bm, _FEAT), lambda i: (i, 0)),
            pl.BlockSpec((_FEAT, 256), lambda i: (0, 0)),
            pl.BlockSpec((1, 256), lambda i: (0, 0)),
            pl.BlockSpec((256, 8), lambda i: (0, 0)),
            pl.BlockSpec((1, 8), lambda i: (0, 0)),
        ],
        out_specs=pl.BlockSpec((bm, 8), lambda i: (i, 0)),
        compiler_params=pltpu.CompilerParams(dimension_semantics=("parallel",)),
    )(person, w1t, b1f, w2p, b2f)


def _tridiag(wt, cin, cout, nw):
    f32 = jnp.float32
    out = jnp.zeros((nw * cin, nw * cout), f32)
    ii = jnp.arange(nw)
    for t in range(3):
        e = ((ii[:, None] - ii[None, :]) == (t - 1)).astype(f32)
        out = out + jnp.kron(e, wt[t].astype(f32))
    return out.astype(jnp.bfloat16)


@jax.jit
def _forward(X, wfa, b2m, w3, b3, w4, b4, w5, b5,
             se16e, se16o, se8e, se8o, w1t, b1f, w2p, b2f):
    f32 = jnp.float32
    x = X.reshape(-1, 2, _NUM_JOINTS, _NUM_ACTORS).astype(f32)
    B = x.shape[0]
    Bp = ((B + _BB - 1) // _BB) * _BB
    nb = Bp // _BB

    xpad = jnp.pad(x, ((0, Bp - B), (0, 0), (1, 8), (0, 0)))
    taps = [xpad[:, kind, kh: kh + 32, :].reshape(Bp, 256)
            for kind in range(2) for kh in range(3)]
    xr = jnp.stack(taps, axis=1)
    xr = xr.reshape(nb, _BB, 6, 256).transpose(0, 2, 1, 3).reshape(nb, 6, _BB * 256)
    xr = jnp.pad(xr, ((0, 0), (0, 2), (0, 0))).astype(jnp.bfloat16)

    wfm = jnp.pad(jnp.transpose(wfa[..., 0], (2, 1, 0)).reshape(32, 6),
                  ((0, 0), (0, 2))).astype(jnp.bfloat16)
    b2t = jnp.tile(jnp.repeat(b2m, 8, axis=1), (1, 8))
    w3b = jnp.stack([_tridiag(w3[:, kh * 32: kh * 32 + 32, :], 32, 64, 8)
                     for kh in range(3)])
    pr = (jnp.arange(256) % 8) * 32 + jnp.arange(256) // 8
    w3b = w3b[:, pr, :]
    w4b = jnp.stack([_tridiag(w4[:, kh * 64: kh * 64 + 64, :], 64, 32, 8)
                     for kh in range(3)])
    w5b = jnp.stack([_tridiag(w5[:, kh * 32: kh * 32 + 32, :], 32, 128, 4)
                     for kh in range(3)])
    b3t = jnp.tile(b3, (1, 8))
    b4t = jnp.tile(b4, (1, 8))
    b5t = jnp.tile(b5, (1, 4))

    feats = _conv_features(xr, wfm, b2t, w3b, b3t, w4b, b4t, w5b, b5t,
                           se16e, se16o, se8e, se8o)
    person = feats.reshape(Bp, _FEAT)
    out = _fc_head(person, w1t, b1f, w2p[:, 0:8], b2f[:, 0:8])
    return out[:B, :_NUM_CLASSES]


def kernel(X, wfa, b2m, w3, b3, w4, b4, w5, b5,
           se16e, se16o, se8e, se8o, w1t, b1f, w2p, b2f):
    return _forward(X, wfa, b2m, w3, b3, w4, b4, w5, b5,
                    se16e, se16o, se8e, se8o, w1t, b1f, w2p, b2f)


# speedup vs baseline: 1.0622x; 1.0355x over previous
import jax
import jax.numpy as jnp
from jax.experimental import pallas as pl
from jax.experimental.pallas import tpu as pltpu

_NUM_JOINTS = 25
_NUM_ACTORS = 8
_NUM_CLASSES = 6
_FEAT = 2048

_BB = 128
_SH = 40
_SH5 = 24
_H0 = 8
_NR = _H0 + _BB * _SH + 8
_NR5 = _H0 + _BB * _SH5 + 8


def _conv_kernel(xr_ref, wfm_ref, b2t_ref, w3b_ref, b3t_ref, w4b_ref, b4t_ref,
                 w5b_ref, b5t_ref, se16e_ref, se16o_ref, se8e_ref, se8o_ref,
                 out_ref, buf3, buf4, buf5):
    f32 = jnp.float32
    bf16 = jnp.bfloat16

    for buf, ss, nv in ((buf3, _SH, 32), (buf4, _SH, 32), (buf5, _SH5, 16)):
        buf[0:_H0, :] = jnp.zeros((_H0, buf.shape[1]), bf16)
        for s in range(_BB):
            r = _H0 + s * ss + nv
            buf[r: r + 8, :] = jnp.zeros((8, buf.shape[1]), bf16)

    for g in range(_BB // 8):
        pa = jnp.dot(wfm_ref[...], xr_ref[0, :, g * 2048: g * 2048 + 2048],
                     preferred_element_type=f32)
        pa = (pa + b2t_ref[...]).astype(bf16)
        for s0 in range(8):
            s = g * 8 + s0
            buf3[_H0 + s * _SH: _H0 + s * _SH + 32, :] = pa[:, s0 * 256: s0 * 256 + 256]

    for c in range(_BB // 4):
        lo = _H0 + c * 4 * _SH
        m = 4 * _SH - 8
        y3 = (jnp.dot(buf3[lo - 1: lo - 1 + m, :], w3b_ref[0],
                      preferred_element_type=f32)
              + jnp.dot(buf3[lo: lo + m, :], w3b_ref[1],
                        preferred_element_type=f32)
              + jnp.dot(buf3[lo + 1: lo + 1 + m, :], w3b_ref[2],
                        preferred_element_type=f32)
              + b3t_ref[...]).astype(bf16)
        for s in range(4):
            buf4[lo + s * _SH: lo + s * _SH + 32, :] = y3[s * _SH: s * _SH + 32, :]

    mcat = []
    for c in range(_BB // 4):
        lo = _H0 + c * 4 * _SH
        m = 4 * _SH - 8
        y4 = (jnp.dot(buf4[lo - 1: lo - 1 + m, :], w4b_ref[0],
                      preferred_element_type=f32)
              + jnp.dot(buf4[lo: lo + m, :], w4b_ref[1],
                        preferred_element_type=f32)
              + jnp.dot(buf4[lo + 1: lo + 1 + m, :], w4b_ref[2],
                        preferred_element_type=f32)
              + b4t_ref[...])
        mw = jnp.concatenate(
            [jnp.maximum(y4[:, 64 * a: 64 * a + 32], y4[:, 64 * a + 32: 64 * a + 64])
             for a in range(4)], axis=1)
        for s in range(4):
            mcat.append(mw[s * _SH: s * _SH + 32, :])
    mcat = jnp.concatenate(mcat, axis=1)
    p4a = jnp.maximum(
        jnp.dot(se16e_ref[...], mcat, preferred_element_type=f32),
        jnp.dot(se16o_ref[...], mcat, preferred_element_type=f32))
    p4a = p4a.astype(bf16)
    for s in range(_BB):
        buf5[_H0 + s * _SH5: _H0 + s * _SH5 + 16, :] = p4a[:, s * 128: s * 128 + 128]

    m5cat = []
    for g in range(_BB // 8):
        lo = _H0 + g * 8 * _SH5
        m = 8 * _SH5 - 8
        y5 = (jnp.dot(buf5[lo - 1: lo - 1 + m, :], w5b_ref[0],
                      preferred_element_type=f32)
              + jnp.dot(buf5[lo: lo + m, :], w5b_ref[1],
                        preferred_element_type=f32)
              + jnp.dot(buf5[lo + 1: lo + 1 + m, :], w5b_ref[2],
                        preferred_element_type=f32)
              + b5t_ref[...])
        m5 = jnp.concatenate([jnp.maximum(y5[:, 0:128], y5[:, 128:256]),
                              jnp.maximum(y5[:, 256:384], y5[:, 384:512])],
                             axis=1)
        m5cat.extend(m5[s * _SH5: s * _SH5 + 16, :] for s in range(8))
    m5cat = jnp.concatenate(m5cat, axis=1)
    p5a = jnp.maximum(
        jnp.dot(se8e_ref[...], m5cat, preferred_element_type=f32),
        jnp.dot(se8o_ref[...], m5cat, preferred_element_type=f32))
    p5a = jnp.maximum(p5a, 0.0).astype(bf16)
    for s in range(_BB):
        for w2 in range(2):
            c0 = s * 256 + w2 * 128
            out_ref[s, w2 * 8: w2 * 8 + 8, :] = p5a[:, c0: c0 + 128]


def _fc_head_kernel(x_ref, w1_ref, b1_ref, w2_ref, b2_ref, o_ref):
    h = jnp.dot(x_ref[...], w1_ref[...], preferred_element_type=jnp.float32)
    h = h + b1_ref[...]
    y = jnp.dot(h, w2_ref[...], preferred_element_type=jnp.float32)
    o_ref[...] = y + b2_ref[...]


def _conv_features(xr, wfm, b2t, w3b, b3t, w4b, b4t, w5b, b5t,
                   se16e, se16o, se8e, se8o):
    nb = xr.shape[0]
    return pl.pallas_call(
        _conv_kernel,
        out_shape=jax.ShapeDtypeStruct((nb * _BB, 16, 128), jnp.bfloat16),
        grid=(nb,),
        in_specs=[
            pl.BlockSpec((1, 8, _BB * 256), lambda i: (i, 0, 0)),
            pl.BlockSpec((32, 8), lambda i: (0, 0)),
            pl.BlockSpec((32, 2048), lambda i: (0, 0)),
            pl.BlockSpec((3, 256, 512), lambda i: (0, 0, 0)),
            pl.BlockSpec((1, 512), lambda i: (0, 0)),
            pl.BlockSpec((3, 512, 256), lambda i: (0, 0, 0)),
            pl.BlockSpec((1, 256), lambda i: (0, 0)),
            pl.BlockSpec((3, 128, 512), lambda i: (0, 0, 0)),
            pl.BlockSpec((1, 512), lambda i: (0, 0)),
            pl.BlockSpec((16, 32), lambda i: (0, 0)),
            pl.BlockSpec((16, 32), lambda i: (0, 0)),
            pl.BlockSpec((8, 16), lambda i: (0, 0)),
            pl.BlockSpec((8, 16), lambda i: (0, 0)),
        ],
        out_specs=pl.BlockSpec((_BB, 16, 128), lambda i: (i, 0, 0)),
        scratch_shapes=[
            pltpu.VMEM((_NR, 256), jnp.bfloat16),
            pltpu.VMEM((_NR, 512), jnp.bfloat16),
            pltpu.VMEM((_NR5, 128), jnp.bfloat16),
        ],
        compiler_params=pltpu.CompilerParams(dimension_semantics=("parallel",)),
    )(xr, wfm, b2t, w3b, b3t, w4b, b4t, w5b, b5t, se16e, se16o, se8e, se8o)


def _fc_head(person, w1t, b1f, w2p, b2f):
    Bp = person.shape[0]
    bm = next(d for d in (256, 128, 64, 32, 16, 8) if Bp % d == 0)
    return pl.pallas_call(
        _fc_head_kernel,
        out_shape=jax.ShapeDtypeStruct((Bp, 8), jnp.float32),
        grid=(Bp // bm,),
        in_specs=[
            pl.BlockSpec((bm, _FEAT), lambda i: (i, 0)),
            pl.BlockSpec((_FEAT, 256), lambda i: (0, 0)),
            pl.BlockSpec((1, 256), lambda i: (0, 0)),
            pl.BlockSpec((256, 8), lambda i: (0, 0)),
            pl.BlockSpec((1, 8), lambda i: (0, 0)),
        ],
        out_specs=pl.BlockSpec((bm, 8), lambda i: (i, 0)),
        compiler_params=pltpu.CompilerParams(dimension_semantics=("parallel",)),
    )(person, w1t, b1f, w2p, b2f)


def _tridiag(wt, cin, cout, nw):
    f32 = jnp.float32
    out = jnp.zeros((nw * cin, nw * cout), f32)
    ii = jnp.arange(nw)
    for t in range(3):
        e = ((ii[:, None] - ii[None, :]) == (t - 1)).astype(f32)
        out = out + jnp.kron(e, wt[t].astype(f32))
    return out.astype(jnp.bfloat16)


@jax.jit
def _forward(X, wfa, b2m, w3, b3, w4, b4, w5, b5,
             se16e, se16o, se8e, se8o, w1t, b1f, w2p, b2f):
    f32 = jnp.float32
    x = X.reshape(-1, 2, _NUM_JOINTS, _NUM_ACTORS).astype(f32)
    B = x.shape[0]
    Bp = ((B + _BB - 1) // _BB) * _BB
    nb = Bp // _BB

    xb = x.astype(jnp.bfloat16)
    xpad = jnp.pad(xb, ((0, Bp - B), (0, 0), (1, 8), (0, 0)))
    taps = [xpad[:, kind, kh: kh + 32, :].reshape(Bp, 256)
            for kind in range(2) for kh in range(3)]
    xr = jnp.stack(taps, axis=1)
    xr = xr.reshape(nb, _BB, 6, 256).transpose(0, 2, 1, 3).reshape(nb, 6, _BB * 256)
    xr = jnp.pad(xr, ((0, 0), (0, 2), (0, 0)))

    wfm = jnp.pad(jnp.transpose(wfa[..., 0], (2, 1, 0)).reshape(32, 6),
                  ((0, 0), (0, 2))).astype(jnp.bfloat16)
    b2t = jnp.tile(jnp.repeat(b2m, 8, axis=1), (1, 8))
    w3b = jnp.stack([_tridiag(w3[:, kh * 32: kh * 32 + 32, :], 32, 64, 8)
                     for kh in range(3)])
    pr = (jnp.arange(256) % 8) * 32 + jnp.arange(256) // 8
    w3b = w3b[:, pr, :]
    w4b = jnp.stack([_tridiag(w4[:, kh * 64: kh * 64 + 64, :], 64, 32, 8)
                     for kh in range(3)])
    w5b = jnp.stack([_tridiag(w5[:, kh * 32: kh * 32 + 32, :], 32, 128, 4)
                     for kh in range(3)])
    b3t = jnp.tile(b3, (1, 8))
    b4t = jnp.tile(b4, (1, 8))
    b5t = jnp.tile(b5, (1, 4))

    feats = _conv_features(xr, wfm, b2t, w3b, b3t, w4b, b4t, w5b, b5t,
                           se16e, se16o, se8e, se8o)
    person = feats.reshape(Bp, _FEAT)
    out = _fc_head(person, w1t.astype(jnp.bfloat16), b1f, w2p[:, 0:8], b2f[:, 0:8])
    return out[:B, :_NUM_CLASSES]


def kernel(X, wfa, b2m, w3, b3, w4, b4, w5, b5,
           se16e, se16o, se8e, se8o, w1t, b1f, w2p, b2f):
    return _forward(X, wfa, b2m, w3, b3, w4, b4, w5, b5,
                    se16e, se16o, se8e, se8o, w1t, b1f, w2p, b2f)
